# baseline (speedup 1.0000x reference)
import functools

import jax
import jax.numpy as jnp
from jax.experimental import pallas as pl
from jax.experimental.pallas import tpu as pltpu

N = 4096
D = 128
V = 1000
NP0 = 8
BLK0 = N // NP0
SLOTS = 2
NP1 = 8
BLK1 = N // NP1
OSLOTS = 2
HALF = N // 2


def _gcn_kernel(a_hbm, x_ref, w1_ref, b1_ref, w2_ref, b2_ref, wd_ref, bd_ref,
                out_hbm, a_bf, z_ref, zb_ref, h1_ref, vin, out_buf,
                sem_in, sem_out):
    s = pl.program_id(0)

    @pl.when(s < NP0)
    def _phase0():
        i = s
        slot = jax.lax.rem(i, SLOTS)

        @pl.when(i == 0)
        def _first():
            for k in range(SLOTS):
                pltpu.make_async_copy(a_hbm.at[pl.ds(k * BLK0, BLK0), :],
                                      vin.at[k], sem_in.at[k]).start()
            z_ref[...] = jnp.dot(x_ref[...], w1_ref[...],
                                 preferred_element_type=jnp.float32)

        @pl.when(jnp.logical_and(i > 0, i + SLOTS - 1 < NP0))
        def _prefetch():
            pf = i + SLOTS - 1
            pltpu.make_async_copy(a_hbm.at[pl.ds(pf * BLK0, BLK0), :],
                                  vin.at[jax.lax.rem(pf, SLOTS)],
                                  sem_in.at[jax.lax.rem(pf, SLOTS)]).start()

        pltpu.make_async_copy(a_hbm.at[pl.ds(i * BLK0, BLK0), :], vin.at[slot],
                              sem_in.at[slot]).wait()
        af = vin[slot]
        a_bf[pl.ds(i * BLK0, BLK0), :] = af.astype(jnp.bfloat16)
        h = jnp.dot(af, z_ref[...], preferred_element_type=jnp.float32)
        h = jnp.maximum(h + b1_ref[...], 0.0)
        h1_ref[pl.ds(i * BLK0, BLK0), :] = h.astype(jnp.bfloat16)

    @pl.when(s >= NP0)
    def _phase1():
        j = s - NP0
        oslot = jax.lax.rem(j, OSLOTS)

        @pl.when(j == 0)
        def _init_z2():
            z2 = jnp.dot(h1_ref[...], w2_ref[...].astype(jnp.bfloat16),
                         preferred_element_type=jnp.float32)
            zb_ref[...] = z2.astype(jnp.bfloat16)

        hu = jnp.dot(a_bf[pl.ds(j * BLK1, BLK1), pl.ds(0, HALF)],
                     zb_ref[pl.ds(0, HALF), :],
                     preferred_element_type=jnp.float32)
        hl = jnp.dot(a_bf[pl.ds(j * BLK1, BLK1), pl.ds(HALF, HALF)],
                     zb_ref[pl.ds(HALF, HALF), :],
                     preferred_element_type=jnp.float32)
        h2 = jnp.maximum(hu + hl + b2_ref[...], 0.0)
        out = jnp.dot(h2.astype(jnp.bfloat16), wd_ref[...].astype(jnp.bfloat16),
                      preferred_element_type=jnp.float32)

        @pl.when(j >= OSLOTS)
        def _wait_prev():
            pltpu.make_async_copy(out_buf.at[oslot],
                                  out_hbm.at[pl.ds((j - OSLOTS) * BLK1, BLK1), :],
                                  sem_out.at[oslot]).wait()

        out_buf[oslot] = out + bd_ref[...]
        pltpu.make_async_copy(out_buf.at[oslot],
                              out_hbm.at[pl.ds(j * BLK1, BLK1), :],
                              sem_out.at[oslot]).start()

        @pl.when(j == NP1 - 1)
        def _drain():
            for k in range(OSLOTS):
                jj = NP1 - OSLOTS + k
                pltpu.make_async_copy(out_buf.at[jax.lax.rem(jj, OSLOTS)],
                                      out_hbm.at[pl.ds(jj * BLK1, BLK1), :],
                                      sem_out.at[jax.lax.rem(jj, OSLOTS)]).wait()


@functools.partial(jax.jit, static_argnames=())
def kernel(feature, graph, W1, b1, W2, b2, Wd, bd):
    b1r = b1.reshape(1, D)
    b2r = b2.reshape(1, D)
    bdr = bd.reshape(1, V)

    out = pl.pallas_call(
        _gcn_kernel,
        grid=(NP0 + NP1,),
        in_specs=[
            pl.BlockSpec(memory_space=pl.ANY),
            pl.BlockSpec((N, D), lambda s: (0, 0)),
            pl.BlockSpec((D, D), lambda s: (0, 0)),
            pl.BlockSpec((1, D), lambda s: (0, 0)),
            pl.BlockSpec((D, D), lambda s: (0, 0)),
            pl.BlockSpec((1, D), lambda s: (0, 0)),
            pl.BlockSpec((D, V), lambda s: (0, 0)),
            pl.BlockSpec((1, V), lambda s: (0, 0)),
        ],
        out_specs=pl.BlockSpec(memory_space=pl.ANY),
        out_shape=jax.ShapeDtypeStruct((N, V), jnp.float32),
        scratch_shapes=[
            pltpu.VMEM((N, N), jnp.bfloat16),
            pltpu.VMEM((N, D), jnp.float32),
            pltpu.VMEM((N, D), jnp.bfloat16),
            pltpu.VMEM((N, D), jnp.bfloat16),
            pltpu.VMEM((SLOTS, BLK0, N), jnp.float32),
            pltpu.VMEM((OSLOTS, BLK1, V), jnp.float32),
            pltpu.SemaphoreType.DMA((SLOTS,)),
            pltpu.SemaphoreType.DMA((OSLOTS,)),
        ],
        compiler_params=pltpu.CompilerParams(
            dimension_semantics=("arbitrary",),
            vmem_limit_bytes=110 * 1024 * 1024,
        ),
    )(graph, feature, W1, b1r, W2, b2r, Wd, bdr)
    return out


# speedup vs baseline: 1.0396x; 1.0396x over previous
import functools

import jax
import jax.numpy as jnp
from jax.experimental import pallas as pl
from jax.experimental.pallas import tpu as pltpu

N = 4096
D = 128
V = 1000
NP0 = 16
BLK0 = N // NP0
SLOTS = 3
NP1 = 8
BLK1 = N // NP1
OSLOTS = 4
HALF = N // 2


def _gcn_kernel(a_hbm, x_ref, w1_ref, b1_ref, w2_ref, b2_ref, wd_ref, bd_ref,
                out_hbm, a_bf, z_ref, zb_ref, vin, out_buf,
                sem_in, sem_out):
    s = pl.program_id(0)

    @pl.when(s < NP0)
    def _phase0():
        i = s
        slot = jax.lax.rem(i, SLOTS)

        @pl.when(i == 0)
        def _first():
            for k in range(SLOTS):
                pltpu.make_async_copy(a_hbm.at[pl.ds(k * BLK0, BLK0), :],
                                      vin.at[k], sem_in.at[k]).start()
            z_ref[...] = jnp.dot(x_ref[...], w1_ref[...],
                                 preferred_element_type=jnp.float32)

        @pl.when(jnp.logical_and(i > 0, i + SLOTS - 1 < NP0))
        def _prefetch():
            pf = i + SLOTS - 1
            pltpu.make_async_copy(a_hbm.at[pl.ds(pf * BLK0, BLK0), :],
                                  vin.at[jax.lax.rem(pf, SLOTS)],
                                  sem_in.at[jax.lax.rem(pf, SLOTS)]).start()

        pltpu.make_async_copy(a_hbm.at[pl.ds(i * BLK0, BLK0), :], vin.at[slot],
                              sem_in.at[slot]).wait()
        af = vin[slot]
        a_bf[pl.ds(i * BLK0, BLK0), :] = af.astype(jnp.bfloat16)
        h = jnp.dot(af, z_ref[...], preferred_element_type=jnp.float32)
        h = jnp.maximum(h + b1_ref[...], 0.0)
        z2 = jnp.dot(h.astype(jnp.bfloat16), w2_ref[...].astype(jnp.bfloat16),
                     preferred_element_type=jnp.float32)
        zb_ref[pl.ds(i * BLK0, BLK0), :] = z2.astype(jnp.bfloat16)

    @pl.when(s >= NP0)
    def _phase1():
        j = s - NP0
        oslot = jax.lax.rem(j, OSLOTS)

        hu = jnp.dot(a_bf[pl.ds(j * BLK1, BLK1), pl.ds(0, HALF)],
                     zb_ref[pl.ds(0, HALF), :],
                     preferred_element_type=jnp.float32)
        hl = jnp.dot(a_bf[pl.ds(j * BLK1, BLK1), pl.ds(HALF, HALF)],
                     zb_ref[pl.ds(HALF, HALF), :],
                     preferred_element_type=jnp.float32)
        h2 = jnp.maximum(hu + hl + b2_ref[...], 0.0)
        out = jnp.dot(h2, wd_ref[...], preferred_element_type=jnp.float32)

        @pl.when(j >= OSLOTS)
        def _wait_prev():
            pltpu.make_async_copy(out_buf.at[oslot],
                                  out_hbm.at[pl.ds((j - OSLOTS) * BLK1, BLK1), :],
                                  sem_out.at[oslot]).wait()

        out_buf[oslot] = out + bd_ref[...]
        pltpu.make_async_copy(out_buf.at[oslot],
                              out_hbm.at[pl.ds(j * BLK1, BLK1), :],
                              sem_out.at[oslot]).start()

        @pl.when(j == NP1 - 1)
        def _drain():
            for k in range(OSLOTS):
                jj = NP1 - OSLOTS + k
                pltpu.make_async_copy(out_buf.at[jax.lax.rem(jj, OSLOTS)],
                                      out_hbm.at[pl.ds(jj * BLK1, BLK1), :],
                                      sem_out.at[jax.lax.rem(jj, OSLOTS)]).wait()


@functools.partial(jax.jit, static_argnames=())
def kernel(feature, graph, W1, b1, W2, b2, Wd, bd):
    b1r = b1.reshape(1, D)
    b2r = b2.reshape(1, D)
    bdr = bd.reshape(1, V)

    out = pl.pallas_call(
        _gcn_kernel,
        grid=(NP0 + NP1,),
        in_specs=[
            pl.BlockSpec(memory_space=pl.ANY),
            pl.BlockSpec((N, D), lambda s: (0, 0)),
            pl.BlockSpec((D, D), lambda s: (0, 0)),
            pl.BlockSpec((1, D), lambda s: (0, 0)),
            pl.BlockSpec((D, D), lambda s: (0, 0)),
            pl.BlockSpec((1, D), lambda s: (0, 0)),
            pl.BlockSpec((D, V), lambda s: (0, 0)),
            pl.BlockSpec((1, V), lambda s: (0, 0)),
        ],
        out_specs=pl.BlockSpec(memory_space=pl.ANY),
        out_shape=jax.ShapeDtypeStruct((N, V), jnp.float32),
        scratch_shapes=[
            pltpu.VMEM((N, N), jnp.bfloat16),
            pltpu.VMEM((N, D), jnp.float32),
            pltpu.VMEM((N, D), jnp.bfloat16),
            pltpu.VMEM((SLOTS, BLK0, N), jnp.float32),
            pltpu.VMEM((OSLOTS, BLK1, V), jnp.float32),
            pltpu.SemaphoreType.DMA((SLOTS,)),
            pltpu.SemaphoreType.DMA((OSLOTS,)),
        ],
        compiler_params=pltpu.CompilerParams(
            dimension_semantics=("arbitrary",),
            vmem_limit_bytes=110 * 1024 * 1024,
        ),
    )(graph, feature, W1, b1r, W2, b2r, Wd, bdr)
    return out
